# baseline (speedup 1.0000x reference)
import functools

import jax
import jax.numpy as jnp
from jax.experimental import pallas as pl
from jax.experimental.pallas import tpu as pltpu


def _fused_body(cuts_ref, x_ref, posT_ref, cen_ref, w_ref,
                W1T_ref, b1_ref, Wa_ref, ba_ref, W2T_ref, b2_ref,
                gamma_ref, beta_ref, W3T_ref, b3_ref, out_ref, acc_ref,
                *, num_tiles, B, C):
    step = pl.program_id(0)
    T = x_ref.shape[0]
    HS = W1T_ref.shape[1]

    @pl.when(step == 0)
    def _init():
        acc_ref[...] = jnp.zeros_like(acc_ref)

    x1 = jax.lax.dot_general(
        x_ref[...], W1T_ref[...], (((1,), (0,)), ((), ())),
        preferred_element_type=jnp.float32) + b1_ref[...]
    x1h = x1.astype(jnp.bfloat16)
    x1l = (x1 - x1h.astype(jnp.float32)).astype(jnp.bfloat16)

    posT = posT_ref[...]
    cen = cen_ref[...]
    d2 = ((cen[:, 0:1] - posT[0:1, :]) ** 2
          + (cen[:, 1:2] - posT[1:2, :]) ** 2
          + (cen[:, 2:3] - posT[2:3, :]) ** 2)
    inv_w2 = 1.0 / (w_ref[...] * w_ref[...])
    rbfT = jnp.exp(-jnp.sqrt(d2) * inv_w2)
    rbf_h = rbfT.astype(jnp.bfloat16)
    rbf_l = (rbfT - rbf_h.astype(jnp.float32)).astype(jnp.bfloat16)

    col = jax.lax.broadcasted_iota(jnp.int32, (1, T), 1)
    base = step * T
    zero = jnp.zeros((), jnp.bfloat16)
    dims = (((1,), (0,)), ((), ()))

    for b in range(B):
        lo = jnp.clip(cuts_ref[b] - base, 0, T)
        hi = jnp.clip(cuts_ref[b + 1] - base, 0, T)

        @pl.when(hi > lo)
        def _seg(lo=lo, hi=hi, b=b):
            rmask = (col >= lo) & (col < hi)
            zh = jnp.where(rmask, rbf_h, zero)
            zl = jnp.where(rmask, rbf_l, zero)
            acc_ref[b * C:(b + 1) * C, :] += (
                jax.lax.dot_general(zh, x1h, dims,
                                    preferred_element_type=jnp.float32)
                + jax.lax.dot_general(zh, x1l, dims,
                                      preferred_element_type=jnp.float32)
                + jax.lax.dot_general(zl, x1h, dims,
                                      preferred_element_type=jnp.float32))

    @pl.when(step == num_tiles - 1)
    def _epilogue():
        feat = acc_ref[...].reshape(B, C, HS)
        featb = feat.astype(jnp.bfloat16).astype(jnp.float32)
        wa = Wa_ref[...].astype(jnp.float32).reshape(1, 1, HS)
        w = jnp.sum(featb * wa, axis=2) + ba_ref[0, 0]
        m = jnp.max(w, axis=1, keepdims=True)
        e = jnp.exp(w - m)
        aw = e / jnp.sum(e, axis=1, keepdims=True)
        agg = jnp.sum(feat * aw[:, :, None], axis=1)
        out = jax.lax.dot_general(
            agg.astype(jnp.bfloat16), W2T_ref[...], (((1,), (0,)), ((), ())),
            preferred_element_type=jnp.float32) + b2_ref[...]
        out = jnp.where(out >= 0, out, 0.2 * out)
        mean = jnp.mean(out, axis=1, keepdims=True)
        cent = out - mean
        var = jnp.mean(cent * cent, axis=1, keepdims=True)
        out = cent / jnp.sqrt(var + 1e-5) * gamma_ref[...] + beta_ref[...]
        out_ref[...] = jax.lax.dot_general(
            out.astype(jnp.bfloat16), W3T_ref[...], (((1,), (0,)), ((), ())),
            preferred_element_type=jnp.float32) + b3_ref[0, 0]


def kernel(x, pos, batch, centers, widths, W1, b1, Wa, ba, W2, b2,
           gamma, beta, W3, b3):
    N, D = x.shape
    HS = W1.shape[0]
    C = centers.shape[0]
    B = 16
    T = 2048
    num_tiles = N // T

    batch_i = batch.astype(jnp.int32)
    cuts = jnp.searchsorted(batch_i, jnp.arange(B + 1, dtype=jnp.int32),
                            side='left').astype(jnp.int32)

    xb = x.astype(jnp.bfloat16)
    posT = pos.T
    w2d = widths.reshape(C, 1)
    W1T = W1.T.astype(jnp.bfloat16)
    b1_2 = b1.reshape(1, HS)
    Wa_2 = Wa.reshape(1, HS).astype(jnp.bfloat16)
    ba_2 = ba.reshape(1, 1)
    W2T = W2.T.astype(jnp.bfloat16)
    b2_2 = b2.reshape(1, D)
    gamma2 = gamma.reshape(1, D)
    beta2 = beta.reshape(1, D)
    W3T = W3.T.astype(jnp.bfloat16)
    b3_2 = b3.reshape(1, 1)

    whole = lambda i, s: (0, 0)
    grid_spec = pltpu.PrefetchScalarGridSpec(
        num_scalar_prefetch=1,
        grid=(num_tiles,),
        in_specs=[
            pl.BlockSpec((T, D), lambda i, s: (i, 0)),
            pl.BlockSpec((3, T), lambda i, s: (0, i)),
            pl.BlockSpec((C, 3), whole),
            pl.BlockSpec((C, 1), whole),
            pl.BlockSpec((D, HS), whole),
            pl.BlockSpec((1, HS), whole),
            pl.BlockSpec((1, HS), whole),
            pl.BlockSpec((1, 1), whole),
            pl.BlockSpec((HS, D), whole),
            pl.BlockSpec((1, D), whole),
            pl.BlockSpec((1, D), whole),
            pl.BlockSpec((1, D), whole),
            pl.BlockSpec((D, 1), whole),
            pl.BlockSpec((1, 1), whole),
        ],
        out_specs=pl.BlockSpec((B, 1), whole),
        scratch_shapes=[pltpu.VMEM((B * C, HS), jnp.float32)],
    )
    out = pl.pallas_call(
        functools.partial(_fused_body, num_tiles=num_tiles, B=B, C=C),
        grid_spec=grid_spec,
        out_shape=jax.ShapeDtypeStruct((B, 1), jnp.float32),
    )(cuts, xb, posT, centers, w2d, W1T, b1_2, Wa_2, ba_2,
      W2T, b2_2, gamma2, beta2, W3T, b3_2)
    return out


# speedup vs baseline: 1.6703x; 1.6703x over previous
import functools

import jax
import jax.numpy as jnp
from jax.experimental import pallas as pl
from jax.experimental.pallas import tpu as pltpu


def _fused_body(x_ref, posT_ref, batchT_ref, segrow_ref, cen_ref, w_ref,
                W1T_ref, b1_ref, Wa_ref, ba_ref, W2T_ref, b2_ref,
                gamma_ref, beta_ref, W3T_ref, b3_ref, out_ref, acc_ref,
                *, num_tiles, B, C):
    step = pl.program_id(0)
    T = x_ref.shape[0]
    D = W1T_ref.shape[0]
    HS = W1T_ref.shape[1]

    @pl.when(step == 0)
    def _init():
        acc_ref[...] = jnp.zeros_like(acc_ref)

    posT = posT_ref[...]
    cen = cen_ref[...]
    d2 = ((cen[:, 0:1] - posT[0:1, :]) ** 2
          + (cen[:, 1:2] - posT[1:2, :]) ** 2
          + (cen[:, 2:3] - posT[2:3, :]) ** 2)
    inv_w2 = 1.0 / (w_ref[...] * w_ref[...])
    rbfT = jnp.exp(-jnp.sqrt(d2) * inv_w2)

    cond = segrow_ref[...] == batchT_ref[...]
    rbf_h = rbfT.astype(jnp.bfloat16)
    rbf_l = (rbfT - rbf_h.astype(jnp.float32)).astype(jnp.bfloat16)
    zero = jnp.zeros((), jnp.bfloat16)
    Rh = jnp.where(cond, jnp.concatenate([rbf_h] * B, axis=0), zero)
    Rl = jnp.where(cond, jnp.concatenate([rbf_l] * B, axis=0), zero)

    dims = (((1,), (0,)), ((), ()))
    xa = x_ref[...]
    acc_ref[...] += (
        jax.lax.dot_general(Rh, xa, dims,
                            preferred_element_type=jnp.float32)
        + jax.lax.dot_general(Rl, xa, dims,
                              preferred_element_type=jnp.float32))

    @pl.when(step == num_tiles - 1)
    def _epilogue():
        G = acc_ref[...]
        Gx = G[:, :D]
        g1 = G[:, D:]
        Gh = Gx.astype(jnp.bfloat16)
        Gl = (Gx - Gh.astype(jnp.float32)).astype(jnp.bfloat16)
        dims2 = (((1,), (0,)), ((), ()))
        featf = (jax.lax.dot_general(Gh, W1T_ref[...], dims2,
                                     preferred_element_type=jnp.float32)
                 + jax.lax.dot_general(Gl, W1T_ref[...], dims2,
                                       preferred_element_type=jnp.float32)
                 + g1 * b1_ref[...])
        feat = featf.reshape(B, C, HS)
        featb = feat.astype(jnp.bfloat16).astype(jnp.float32)
        wa = Wa_ref[...].astype(jnp.float32).reshape(1, 1, HS)
        w = jnp.sum(featb * wa, axis=2) + ba_ref[0, 0]
        m = jnp.max(w, axis=1, keepdims=True)
        e = jnp.exp(w - m)
        aw = e / jnp.sum(e, axis=1, keepdims=True)
        agg = jnp.sum(feat * aw[:, :, None], axis=1)
        out = jax.lax.dot_general(
            agg.astype(jnp.bfloat16), W2T_ref[...], (((1,), (0,)), ((), ())),
            preferred_element_type=jnp.float32) + b2_ref[...]
        out = jnp.where(out >= 0, out, 0.2 * out)
        mean = jnp.mean(out, axis=1, keepdims=True)
        cent = out - mean
        var = jnp.mean(cent * cent, axis=1, keepdims=True)
        out = cent / jnp.sqrt(var + 1e-5) * gamma_ref[...] + beta_ref[...]
        out_ref[...] = jax.lax.dot_general(
            out.astype(jnp.bfloat16), W3T_ref[...], (((1,), (0,)), ((), ())),
            preferred_element_type=jnp.float32) + b3_ref[0, 0]


def kernel(x, pos, batch, centers, widths, W1, b1, Wa, ba, W2, b2,
           gamma, beta, W3, b3):
    N, D = x.shape
    HS = W1.shape[0]
    C = centers.shape[0]
    B = 16
    T = 2048
    num_tiles = N // T

    xaug = jnp.concatenate(
        [x, jnp.ones((N, 1), x.dtype)], axis=1).astype(jnp.bfloat16)
    posT = pos.T
    batchT = batch.astype(jnp.bfloat16).reshape(1, N)
    segrow = (jnp.arange(B * C, dtype=jnp.float32)
              // C).astype(jnp.bfloat16).reshape(B * C, 1)
    w2d = widths.reshape(C, 1)
    W1T = W1.T.astype(jnp.bfloat16)
    b1_2 = b1.reshape(1, HS)
    Wa_2 = Wa.reshape(1, HS).astype(jnp.bfloat16)
    ba_2 = ba.reshape(1, 1)
    W2T = W2.T.astype(jnp.bfloat16)
    b2_2 = b2.reshape(1, D)
    gamma2 = gamma.reshape(1, D)
    beta2 = beta.reshape(1, D)
    W3T = W3.T.astype(jnp.bfloat16)
    b3_2 = b3.reshape(1, 1)

    whole = lambda i: (0, 0)
    out = pl.pallas_call(
        functools.partial(_fused_body, num_tiles=num_tiles, B=B, C=C),
        grid=(num_tiles,),
        in_specs=[
            pl.BlockSpec((T, D + 1), lambda i: (i, 0)),
            pl.BlockSpec((3, T), lambda i: (0, i)),
            pl.BlockSpec((1, T), lambda i: (0, i)),
            pl.BlockSpec((B * C, 1), whole),
            pl.BlockSpec((C, 3), whole),
            pl.BlockSpec((C, 1), whole),
            pl.BlockSpec((D, HS), whole),
            pl.BlockSpec((1, HS), whole),
            pl.BlockSpec((1, HS), whole),
            pl.BlockSpec((1, 1), whole),
            pl.BlockSpec((HS, D), whole),
            pl.BlockSpec((1, D), whole),
            pl.BlockSpec((1, D), whole),
            pl.BlockSpec((1, D), whole),
            pl.BlockSpec((D, 1), whole),
            pl.BlockSpec((1, 1), whole),
        ],
        out_specs=pl.BlockSpec((B, 1), whole),
        out_shape=jax.ShapeDtypeStruct((B, 1), jnp.float32),
        scratch_shapes=[pltpu.VMEM((B * C, D + 1), jnp.float32)],
    )(xaug, posT, batchT, segrow, centers, w2d, W1T, b1_2, Wa_2, ba_2,
      W2T, b2_2, gamma2, beta2, W3T, b3_2)
    return out


# speedup vs baseline: 1.7375x; 1.0402x over previous
import functools

import jax
import jax.numpy as jnp
from jax.experimental import pallas as pl
from jax.experimental.pallas import tpu as pltpu


def _fused_body(x_ref, posT_ref, batchT_ref, cen_ref, w_ref,
                W1T_ref, b1_ref, Wa_ref, ba_ref, W2T_ref, b2_ref,
                gamma_ref, beta_ref, W3T_ref, b3_ref, out_ref, acc_ref,
                *, num_tiles, B, C):
    step = pl.program_id(0)
    T = x_ref.shape[0]
    D = W1T_ref.shape[0]
    HS = W1T_ref.shape[1]

    @pl.when(step == 0)
    def _init():
        acc_ref[...] = jnp.zeros_like(acc_ref)

    posT = posT_ref[...]
    cen = cen_ref[...]
    d2 = ((cen[:, 0:1] - posT[0:1, :]) ** 2
          + (cen[:, 1:2] - posT[1:2, :]) ** 2
          + (cen[:, 2:3] - posT[2:3, :]) ** 2)
    inv_w2 = 1.0 / (w_ref[...] * w_ref[...])
    rbfT = jnp.exp(-jnp.sqrt(d2) * inv_w2)

    bids = jax.lax.broadcasted_iota(
        jnp.int32, (B, 1, 1), 0).astype(jnp.bfloat16)
    cond3 = bids == batchT_ref[...].reshape(1, 1, T)
    rbf_h = rbfT.astype(jnp.bfloat16)
    rbf_l = (rbfT - rbf_h.astype(jnp.float32)).astype(jnp.bfloat16)
    zero = jnp.zeros((), jnp.bfloat16)
    Rh = jnp.where(cond3, rbf_h[None], zero).reshape(B * C, T)
    Rl = jnp.where(cond3, rbf_l[None], zero).reshape(B * C, T)

    dims = (((1,), (0,)), ((), ()))
    xa = x_ref[...]
    acc_ref[...] += (
        jax.lax.dot_general(Rh, xa, dims,
                            preferred_element_type=jnp.float32)
        + jax.lax.dot_general(Rl, xa, dims,
                              preferred_element_type=jnp.float32))

    @pl.when(step == num_tiles - 1)
    def _epilogue():
        G = acc_ref[...]
        Gx = G[:, :D]
        g1 = G[:, D:]
        Gh = Gx.astype(jnp.bfloat16)
        Gl = (Gx - Gh.astype(jnp.float32)).astype(jnp.bfloat16)
        dims2 = (((1,), (0,)), ((), ()))
        featf = (jax.lax.dot_general(Gh, W1T_ref[...], dims2,
                                     preferred_element_type=jnp.float32)
                 + jax.lax.dot_general(Gl, W1T_ref[...], dims2,
                                       preferred_element_type=jnp.float32)
                 + g1 * b1_ref[...])
        feat = featf.reshape(B, C, HS)
        featb = feat.astype(jnp.bfloat16).astype(jnp.float32)
        wa = Wa_ref[...].astype(jnp.float32).reshape(1, 1, HS)
        w = jnp.sum(featb * wa, axis=2) + ba_ref[0, 0]
        m = jnp.max(w, axis=1, keepdims=True)
        e = jnp.exp(w - m)
        aw = e / jnp.sum(e, axis=1, keepdims=True)
        agg = jnp.sum(feat * aw[:, :, None], axis=1)
        out = jax.lax.dot_general(
            agg.astype(jnp.bfloat16), W2T_ref[...], (((1,), (0,)), ((), ())),
            preferred_element_type=jnp.float32) + b2_ref[...]
        out = jnp.where(out >= 0, out, 0.2 * out)
        mean = jnp.mean(out, axis=1, keepdims=True)
        cent = out - mean
        var = jnp.mean(cent * cent, axis=1, keepdims=True)
        out = cent / jnp.sqrt(var + 1e-5) * gamma_ref[...] + beta_ref[...]
        out_ref[...] = jax.lax.dot_general(
            out.astype(jnp.bfloat16), W3T_ref[...], (((1,), (0,)), ((), ())),
            preferred_element_type=jnp.float32) + b3_ref[0, 0]


def kernel(x, pos, batch, centers, widths, W1, b1, Wa, ba, W2, b2,
           gamma, beta, W3, b3):
    N, D = x.shape
    HS = W1.shape[0]
    C = centers.shape[0]
    B = 16
    T = 2048
    num_tiles = N // T

    xaug = jnp.concatenate(
        [x, jnp.ones((N, 1), x.dtype)], axis=1).astype(jnp.bfloat16)
    posT = pos.T
    batchT = batch.astype(jnp.bfloat16).reshape(1, N)
    w2d = widths.reshape(C, 1)
    W1T = W1.T.astype(jnp.bfloat16)
    b1_2 = b1.reshape(1, HS)
    Wa_2 = Wa.reshape(1, HS).astype(jnp.bfloat16)
    ba_2 = ba.reshape(1, 1)
    W2T = W2.T.astype(jnp.bfloat16)
    b2_2 = b2.reshape(1, D)
    gamma2 = gamma.reshape(1, D)
    beta2 = beta.reshape(1, D)
    W3T = W3.T.astype(jnp.bfloat16)
    b3_2 = b3.reshape(1, 1)

    whole = lambda i: (0, 0)
    out = pl.pallas_call(
        functools.partial(_fused_body, num_tiles=num_tiles, B=B, C=C),
        grid=(num_tiles,),
        in_specs=[
            pl.BlockSpec((T, D + 1), lambda i: (i, 0)),
            pl.BlockSpec((3, T), lambda i: (0, i)),
            pl.BlockSpec((1, T), lambda i: (0, i)),
            pl.BlockSpec((C, 3), whole),
            pl.BlockSpec((C, 1), whole),
            pl.BlockSpec((D, HS), whole),
            pl.BlockSpec((1, HS), whole),
            pl.BlockSpec((1, HS), whole),
            pl.BlockSpec((1, 1), whole),
            pl.BlockSpec((HS, D), whole),
            pl.BlockSpec((1, D), whole),
            pl.BlockSpec((1, D), whole),
            pl.BlockSpec((1, D), whole),
            pl.BlockSpec((D, 1), whole),
            pl.BlockSpec((1, 1), whole),
        ],
        out_specs=pl.BlockSpec((B, 1), whole),
        out_shape=jax.ShapeDtypeStruct((B, 1), jnp.float32),
        scratch_shapes=[pltpu.VMEM((B * C, D + 1), jnp.float32)],
    )(xaug, posT, batchT, centers, w2d, W1T, b1_2, Wa_2, ba_2,
      W2T, b2_2, gamma2, beta2, W3T, b3_2)
    return out


# speedup vs baseline: 1.8597x; 1.0704x over previous
import functools

import jax
import jax.numpy as jnp
from jax.experimental import pallas as pl
from jax.experimental.pallas import tpu as pltpu


def _fused_body(x_ref, posT_ref, batchT_ref, cen_ref, w_ref,
                W1T_ref, b1_ref, Wa_ref, ba_ref, W2T_ref, b2_ref,
                gamma_ref, beta_ref, W3T_ref, b3_ref, out_ref, acc_ref,
                *, num_tiles, B, C):
    step = pl.program_id(0)
    T = x_ref.shape[0]
    D = W1T_ref.shape[0]
    HS = W1T_ref.shape[1]

    @pl.when(step == 0)
    def _init():
        acc_ref[...] = jnp.zeros_like(acc_ref)

    posT = posT_ref[...]
    cen = cen_ref[...]
    d2 = ((cen[:, 0:1] - posT[0:1, :]) ** 2
          + (cen[:, 1:2] - posT[1:2, :]) ** 2
          + (cen[:, 2:3] - posT[2:3, :]) ** 2)
    inv_w2 = 1.0 / (w_ref[...] * w_ref[...])
    rbfT = jnp.exp(-jnp.sqrt(d2) * inv_w2)

    bids = jax.lax.broadcasted_iota(
        jnp.int32, (B, 1, 1), 0).astype(jnp.bfloat16)
    cond3 = bids == batchT_ref[...].reshape(1, 1, T)
    rbf_h = rbfT.astype(jnp.bfloat16)
    rbf_l = (rbfT - rbf_h.astype(jnp.float32)).astype(jnp.bfloat16)
    zero = jnp.zeros((), jnp.bfloat16)
    Rh = jnp.where(cond3, rbf_h[None], zero).reshape(B * C, T)
    Rl = jnp.where(cond3, rbf_l[None], zero).reshape(B * C, T)

    dims = (((1,), (0,)), ((), ()))
    xa = x_ref[...]
    acc_ref[...] += (
        jax.lax.dot_general(Rh, xa, dims,
                            preferred_element_type=jnp.float32)
        + jax.lax.dot_general(Rl, xa, dims,
                              preferred_element_type=jnp.float32))

    @pl.when(step == num_tiles - 1)
    def _epilogue():
        G = acc_ref[...]
        Gx = G[:, :D]
        g1 = G[:, D:]
        Gh = Gx.astype(jnp.bfloat16)
        Gl = (Gx - Gh.astype(jnp.float32)).astype(jnp.bfloat16)
        dims2 = (((1,), (0,)), ((), ()))
        featf = (jax.lax.dot_general(Gh, W1T_ref[...], dims2,
                                     preferred_element_type=jnp.float32)
                 + jax.lax.dot_general(Gl, W1T_ref[...], dims2,
                                       preferred_element_type=jnp.float32)
                 + g1 * b1_ref[...])
        feat = featf.reshape(B, C, HS)
        featb = feat.astype(jnp.bfloat16).astype(jnp.float32)
        wa = Wa_ref[...].astype(jnp.float32).reshape(1, 1, HS)
        w = jnp.sum(featb * wa, axis=2) + ba_ref[0, 0]
        m = jnp.max(w, axis=1, keepdims=True)
        e = jnp.exp(w - m)
        aw = e / jnp.sum(e, axis=1, keepdims=True)
        agg = jnp.sum(feat * aw[:, :, None], axis=1)
        out = jax.lax.dot_general(
            agg.astype(jnp.bfloat16), W2T_ref[...], (((1,), (0,)), ((), ())),
            preferred_element_type=jnp.float32) + b2_ref[...]
        out = jnp.where(out >= 0, out, 0.2 * out)
        mean = jnp.mean(out, axis=1, keepdims=True)
        cent = out - mean
        var = jnp.mean(cent * cent, axis=1, keepdims=True)
        out = cent / jnp.sqrt(var + 1e-5) * gamma_ref[...] + beta_ref[...]
        out_ref[...] = jax.lax.dot_general(
            out.astype(jnp.bfloat16), W3T_ref[...], (((1,), (0,)), ((), ())),
            preferred_element_type=jnp.float32) + b3_ref[0, 0]


def kernel(x, pos, batch, centers, widths, W1, b1, Wa, ba, W2, b2,
           gamma, beta, W3, b3):
    N, D = x.shape
    HS = W1.shape[0]
    C = centers.shape[0]
    B = 16
    T = 4096
    num_tiles = N // T

    xaug = jnp.concatenate(
        [x, jnp.ones((N, 1), x.dtype)], axis=1).astype(jnp.bfloat16)
    posT = pos.T
    batchT = batch.astype(jnp.bfloat16).reshape(1, N)
    w2d = widths.reshape(C, 1)
    W1T = W1.T.astype(jnp.bfloat16)
    b1_2 = b1.reshape(1, HS)
    Wa_2 = Wa.reshape(1, HS).astype(jnp.bfloat16)
    ba_2 = ba.reshape(1, 1)
    W2T = W2.T.astype(jnp.bfloat16)
    b2_2 = b2.reshape(1, D)
    gamma2 = gamma.reshape(1, D)
    beta2 = beta.reshape(1, D)
    W3T = W3.T.astype(jnp.bfloat16)
    b3_2 = b3.reshape(1, 1)

    whole = lambda i: (0, 0)
    out = pl.pallas_call(
        functools.partial(_fused_body, num_tiles=num_tiles, B=B, C=C),
        grid=(num_tiles,),
        in_specs=[
            pl.BlockSpec((T, D + 1), lambda i: (i, 0)),
            pl.BlockSpec((3, T), lambda i: (0, i)),
            pl.BlockSpec((1, T), lambda i: (0, i)),
            pl.BlockSpec((C, 3), whole),
            pl.BlockSpec((C, 1), whole),
            pl.BlockSpec((D, HS), whole),
            pl.BlockSpec((1, HS), whole),
            pl.BlockSpec((1, HS), whole),
            pl.BlockSpec((1, 1), whole),
            pl.BlockSpec((HS, D), whole),
            pl.BlockSpec((1, D), whole),
            pl.BlockSpec((1, D), whole),
            pl.BlockSpec((1, D), whole),
            pl.BlockSpec((D, 1), whole),
            pl.BlockSpec((1, 1), whole),
        ],
        out_specs=pl.BlockSpec((B, 1), whole),
        out_shape=jax.ShapeDtypeStruct((B, 1), jnp.float32),
        scratch_shapes=[pltpu.VMEM((B * C, D + 1), jnp.float32)],
    )(xaug, posT, batchT, centers, w2d, W1T, b1_2, Wa_2, ba_2,
      W2T, b2_2, gamma2, beta2, W3T, b3_2)
    return out
